# Initial kernel scaffold; baseline (speedup 1.0000x reference)
import jax
import jax.numpy as jnp
from jax.experimental import pallas as pl


def kernel(x_disease, x_protein, assoc_src, assoc_dst, inter_src, inter_dst, W_self, W_neigh, b_conv, bn_gamma, bn_beta, W_proj, b_proj):
    raise NotImplementedError("write your pallas kernel here")


# speedup vs baseline: 2.7305x; 2.7305x over previous
import functools

import jax
import jax.numpy as jnp
from jax import lax
from jax.experimental import pallas as pl
from jax.experimental.pallas import tpu as pltpu
from jax.experimental.pallas import tpu_sc as plsc

N_D = 10000
N_P = 50000
D = 128
H = 128
EPS = 1e-5

NC = 2
NS = 16
BD = 128
CH = 32
NCH = D // CH
ZR = 256
KB = 8

N_P_PAD = 50048
N_D_PAD = 10112

RT = 2000


def _ceil_to(x, m):
    return ((x + m - 1) // m) * m


def _sc_mesh():
    return plsc.VectorSubcoreMesh(core_axis_name="c", subcore_axis_name="s",
                                  num_cores=NC, num_subcores=NS)


def _emit_zero(spmem, zv, base, stripe, width):
    full, rem = divmod(stripe, ZR)
    if full:
        @pl.loop(0, full)
        def _z(z):
            pltpu.sync_copy(zv, spmem.at[pl.ds(base + z * ZR, ZR)])
    if rem:
        pltpu.sync_copy(zv.at[pl.ds(0, rem)],
                        spmem.at[pl.ds(base + full * ZR, rem)])


def _make_agg(ea_pad, ei_pad):
    steps_a = ea_pad // (NS * BD)
    steps_i = ei_pad // (NS * BD)
    mesh = _sc_mesh()

    out_type = (
        [jax.ShapeDtypeStruct((N_P_PAD, CH), jnp.float32)] * NCH
        + [jax.ShapeDtypeStruct((N_D_PAD, CH), jnp.float32)] * NCH
        + [jax.ShapeDtypeStruct((N_P_PAD, CH), jnp.float32)] * NCH
        + [jax.ShapeDtypeStruct((N_P_PAD, CH), jnp.float32)] * NCH
    )
    scratch = [
        pltpu.VMEM_SHARED((N_P_PAD, CH), jnp.float32),
        pltpu.VMEM((KB, BD), jnp.int32),
        pltpu.VMEM((KB, BD), jnp.int32),
        pltpu.VMEM((BD, CH), jnp.float32),
        pltpu.VMEM((BD, CH), jnp.float32),
        pltpu.VMEM((ZR, CH), jnp.float32),
        pltpu.SemaphoreType.DMA,
        pltpu.SemaphoreType.DMA,
    ]

    @functools.partial(pl.kernel, out_type=out_type, mesh=mesh,
                       scratch_types=scratch,
                       compiler_params=pltpu.CompilerParams(
                           use_tc_tiling_on_sc=False))
    def agg(zeros_hbm, hd0, hd1, hd2, hd3, hp0, hp1, hp2, hp3,
            g1, s1, g2, s2, g3, s3, g4, s4, *rest):
        outs = rest[:4 * NCH]
        spmem, gv, sv, rows0, rows1, zv, sem0, sem1 = rest[4 * NCH:]
        core = lax.axis_index("c")
        tile = lax.axis_index("s")
        pltpu.sync_copy(zeros_hbm, zv)

        hd = (hd0, hd1, hd2, hd3)
        hp = (hp0, hp1, hp2, hp3)
        aggs = [
            (g1, s1, hd, outs[0:4], N_P_PAD, steps_a),
            (g2, s2, hp, outs[4:8], N_D_PAD, steps_a),
            (g3, s3, hp, outs[8:12], N_P_PAD, steps_i),
            (g4, s4, hp, outs[12:16], N_P_PAD, steps_i),
        ]

        def emit_task(g_hbm, s_hbm, tbl, out_hbm, npad, steps):
            stripe = npad // NS
            base = tile * stripe
            _emit_zero(spmem, zv, base, stripe, CH)
            plsc.subcore_barrier()

            @pl.loop(0, steps // KB)
            def _b(bk):
                r0 = tile * steps + bk * KB
                pltpu.sync_copy(g_hbm.at[pl.ds(r0, KB)], gv)
                pltpu.sync_copy(s_hbm.at[pl.ds(r0, KB)], sv)
                pltpu.async_copy(tbl.at[gv.at[0]], rows0, sem0)
                for j in range(0, KB, 2):
                    pltpu.make_async_copy(tbl.at[gv.at[j]], rows0,
                                          sem0).wait()
                    pltpu.async_copy(tbl.at[gv.at[j + 1]], rows1, sem1)
                    pltpu.sync_copy(rows0, spmem.at[sv.at[j]], add=True)
                    if j + 2 < KB:
                        pltpu.async_copy(tbl.at[gv.at[j + 2]], rows0, sem0)
                    pltpu.make_async_copy(tbl.at[gv.at[j + 1]], rows1,
                                          sem1).wait()
                    pltpu.sync_copy(rows1, spmem.at[sv.at[j + 1]], add=True)

            plsc.subcore_barrier()
            pltpu.sync_copy(spmem.at[pl.ds(base, stripe)],
                            out_hbm.at[pl.ds(base, stripe)])
            plsc.subcore_barrier()

        for g_hbm, s_hbm, tbls, outs4, npad, steps in aggs:
            for ch in range(NCH):
                which = ch % 2

                @pl.when(core == which)
                def _(g_hbm=g_hbm, s_hbm=s_hbm, tbl=tbls[ch],
                      out_hbm=outs4[ch], npad=npad, steps=steps):
                    emit_task(g_hbm, s_hbm, tbl, out_hbm, npad, steps)

    return agg


def _make_counts(ea_pad, ei_pad):
    steps_a = ea_pad // (NS * BD)
    steps_i = ei_pad // (NS * BD)
    max_steps = max(steps_a, steps_i)
    mesh = _sc_mesh()

    out_type = [
        jax.ShapeDtypeStruct((N_P_PAD, 16), jnp.float32),
        jax.ShapeDtypeStruct((N_D_PAD, 16), jnp.float32),
        jax.ShapeDtypeStruct((N_P_PAD, 16), jnp.float32),
        jax.ShapeDtypeStruct((N_P_PAD, 16), jnp.float32),
    ]
    scratch = [
        pltpu.VMEM_SHARED((N_P_PAD, 16), jnp.float32),
        pltpu.VMEM((max_steps, BD), jnp.int32),
        pltpu.VMEM((BD, 16), jnp.float32),
        pltpu.VMEM((ZR, 16), jnp.float32),
    ]

    @functools.partial(pl.kernel, out_type=out_type, mesh=mesh,
                       scratch_types=scratch,
                       compiler_params=pltpu.CompilerParams(
                           use_tc_tiling_on_sc=False))
    def counts(ones_hbm, zeros_hbm, s1, s2, s3, s4,
               c1, c2, c3, c4, spmem, sv, onev, zv):
        core = lax.axis_index("c")
        tile = lax.axis_index("s")
        pltpu.sync_copy(ones_hbm, onev)
        pltpu.sync_copy(zeros_hbm, zv)

        def emit_task(s_hbm, out_hbm, npad, steps):
            stripe = npad // NS
            base = tile * stripe
            _emit_zero(spmem, zv, base, stripe, 16)
            r0 = tile * steps
            pltpu.sync_copy(s_hbm.at[pl.ds(r0, steps)],
                            sv.at[pl.ds(0, steps)])
            plsc.subcore_barrier()

            @pl.loop(0, steps)
            def _e(j):
                pltpu.sync_copy(onev, spmem.at[sv.at[j]], add=True)

            plsc.subcore_barrier()
            pltpu.sync_copy(spmem.at[pl.ds(base, stripe)],
                            out_hbm.at[pl.ds(base, stripe)])
            plsc.subcore_barrier()

        tasks = [
            (s3, c3, N_P_PAD, steps_i, 0),
            (s2, c2, N_D_PAD, steps_a, 0),
            (s1, c1, N_P_PAD, steps_a, 1),
            (s4, c4, N_P_PAD, steps_i, 1),
        ]
        for s_hbm, out_hbm, npad, steps, which in tasks:
            @pl.when(core == which)
            def _(s_hbm=s_hbm, out_hbm=out_hbm, npad=npad, steps=steps):
                emit_task(s_hbm, out_hbm, npad, steps)

    return counts


def _conv_stats(h, groups, w_big, bias, n_rows):
    t = n_rows // RT
    n_g = len(groups)

    def body(*refs):
        h_ref = refs[0]
        pos = 1
        parts = [h_ref[...]]
        for _ in range(n_g):
            s_refs = refs[pos:pos + NCH]
            cnt_ref = refs[pos + NCH]
            pos += NCH + 1
            inv = 1.0 / jnp.maximum(cnt_ref[:, 0:1], 1.0)
            for c in range(NCH):
                parts.append(s_refs[c][...] * inv)
        w_ref, b_ref = refs[pos], refs[pos + 1]
        conv_ref, stats_ref, acc_ref = refs[pos + 2:pos + 5]
        x = jnp.concatenate(parts, axis=1)
        y = jnp.dot(x, w_ref[...], preferred_element_type=jnp.float32)
        y = y + b_ref[...]
        conv_ref[...] = y
        i = pl.program_id(0)

        @pl.when(i == 0)
        def _():
            acc_ref[...] = jnp.zeros_like(acc_ref)

        acc_ref[0:1, :] += jnp.sum(y, axis=0)[None, :]
        acc_ref[1:2, :] += jnp.sum(y * y, axis=0)[None, :]
        stats_ref[...] = acc_ref[...]

    in_specs = [pl.BlockSpec((RT, D), lambda i: (i, 0))]
    args = [h]
    for s_chunks, cnt in groups:
        for c in range(NCH):
            in_specs.append(pl.BlockSpec((RT, CH), lambda i: (i, 0)))
            args.append(s_chunks[c])
        in_specs.append(pl.BlockSpec((RT, 16), lambda i: (i, 0)))
        args.append(cnt)
    k_dim = (1 + n_g) * D
    in_specs.append(pl.BlockSpec((k_dim, H), lambda i: (0, 0)))
    args.append(w_big)
    in_specs.append(pl.BlockSpec((1, H), lambda i: (0, 0)))
    args.append(bias)

    conv, stats = pl.pallas_call(
        body,
        grid=(t,),
        in_specs=in_specs,
        out_specs=[pl.BlockSpec((RT, H), lambda i: (i, 0)),
                   pl.BlockSpec((8, H), lambda i: (0, 0))],
        out_shape=[jax.ShapeDtypeStruct((n_rows, H), jnp.float32),
                   jax.ShapeDtypeStruct((8, H), jnp.float32)],
        scratch_shapes=[pltpu.VMEM((8, H), jnp.float32)],
    )(*args)
    return conv, stats


def _norm(conv, stats, gamma, beta, n_rows, make_chunks, w_p=None, b_p=None):
    t = n_rows // RT
    proj = w_p is not None

    def body(*refs):
        conv_ref, stats_ref, g_ref, be_ref = refs[:4]
        pos = 4
        if proj:
            wp_ref, bp_ref = refs[pos:pos + 2]
            pos += 2
        outs = refs[pos:]
        n = float(n_rows)
        mu = stats_ref[0:1, :] / n
        var = stats_ref[1:2, :] / n - mu * mu
        scale = g_ref[...] * lax.rsqrt(var + EPS)
        shift = be_ref[...] - mu * scale
        y = jnp.maximum(conv_ref[...] * scale + shift, 0.0)
        if proj:
            outs[0][...] = (jnp.dot(y, wp_ref[...],
                                    preferred_element_type=jnp.float32)
                            + bp_ref[...])
        else:
            outs[0][...] = y
            if make_chunks:
                for c in range(NCH):
                    outs[1 + c][...] = y[:, c * CH:(c + 1) * CH]

    in_specs = [pl.BlockSpec((RT, H), lambda i: (i, 0)),
                pl.BlockSpec((8, H), lambda i: (0, 0)),
                pl.BlockSpec((1, H), lambda i: (0, 0)),
                pl.BlockSpec((1, H), lambda i: (0, 0))]
    args = [conv, stats, gamma, beta]
    if proj:
        in_specs += [pl.BlockSpec((H, D), lambda i: (0, 0)),
                     pl.BlockSpec((1, D), lambda i: (0, 0))]
        args += [w_p, b_p]

    out_specs = [pl.BlockSpec((RT, H), lambda i: (i, 0))]
    out_shape = [jax.ShapeDtypeStruct((n_rows, H), jnp.float32)]
    if (not proj) and make_chunks:
        for c in range(NCH):
            out_specs.append(pl.BlockSpec((RT, CH), lambda i: (i, 0)))
            out_shape.append(jax.ShapeDtypeStruct((n_rows, CH), jnp.float32))

    res = pl.pallas_call(
        body,
        grid=(t,),
        in_specs=in_specs,
        out_specs=out_specs,
        out_shape=out_shape,
    )(*args)
    return res


def _pad_idx(idx, e_pad, dummy_base):
    pad = e_pad - idx.shape[0]
    idx = idx.astype(jnp.int32)
    if dummy_base is None:
        tail = jnp.zeros((pad,), jnp.int32)
    else:
        tail = dummy_base + (jnp.arange(pad, dtype=jnp.int32) % 8)
    return jnp.concatenate([idx, tail]).reshape(e_pad // BD, BD)


def kernel(x_disease, x_protein, assoc_src, assoc_dst, inter_src, inter_dst,
           W_self, W_neigh, b_conv, bn_gamma, bn_beta, W_proj, b_proj):
    e_a = assoc_src.shape[0]
    e_i = inter_src.shape[0]
    ea_pad = _ceil_to(e_a, NS * BD * 8)
    ei_pad = _ceil_to(e_i, NS * BD * 8)

    g1 = _pad_idx(assoc_src, ea_pad, None)
    s1 = _pad_idx(assoc_dst, ea_pad, N_P)
    g2 = _pad_idx(assoc_dst, ea_pad, None)
    s2 = _pad_idx(assoc_src, ea_pad, N_D)
    g3 = _pad_idx(inter_src, ei_pad, None)
    s3 = _pad_idx(inter_dst, ei_pad, N_P)
    g4 = _pad_idx(inter_dst, ei_pad, None)
    s4 = _pad_idx(inter_src, ei_pad, N_P)

    zeros32 = jnp.zeros((ZR, CH), jnp.float32)
    zeros16 = jnp.zeros((ZR, 16), jnp.float32)
    ones16 = jnp.ones((BD, 16), jnp.float32)

    agg = _make_agg(ea_pad, ei_pad)
    counts = _make_counts(ea_pad, ei_pad)

    c1, c2, c3, c4 = counts(ones16, zeros16, s1, s2, s3, s4)

    h_d, h_p = x_disease, x_protein
    hd_ch = [x_disease[:, c * CH:(c + 1) * CH] for c in range(NCH)]
    hp_ch = [x_protein[:, c * CH:(c + 1) * CH] for c in range(NCH)]

    out_d = out_p = None
    for i in range(2):
        outs = agg(zeros32, *hd_ch, *hp_ch, g1, s1, g2, s2, g3, s3, g4, s4)
        S1 = outs[0:4]
        S2 = outs[4:8]
        S3 = outs[8:12]
        S4 = outs[12:16]

        w_big_p = jnp.concatenate(
            [W_self[i, 0] + W_self[i, 2] + W_self[i, 3],
             W_neigh[i, 0], W_neigh[i, 2], W_neigh[i, 3]], axis=0)
        b_p_sum = (b_conv[i, 0] + b_conv[i, 2] + b_conv[i, 3]).reshape(1, H)
        w_big_d = jnp.concatenate([W_self[i, 1], W_neigh[i, 1]], axis=0)
        b_d = b_conv[i, 1].reshape(1, H)

        conv_p, stats_p = _conv_stats(
            h_p, [(S1, c1), (S3, c3), (S4, c4)], w_big_p, b_p_sum, N_P)
        conv_d, stats_d = _conv_stats(h_d, [(S2, c2)], w_big_d, b_d, N_D)

        g_d = bn_gamma[i, 0].reshape(1, H)
        be_d = bn_beta[i, 0].reshape(1, H)
        g_p = bn_gamma[i, 1].reshape(1, H)
        be_p = bn_beta[i, 1].reshape(1, H)

        if i == 0:
            res_d = _norm(conv_d, stats_d, g_d, be_d, N_D, True)
            h_d, hd_ch = res_d[0], list(res_d[1:])
            res_p = _norm(conv_p, stats_p, g_p, be_p, N_P, True)
            h_p, hp_ch = res_p[0], list(res_p[1:])
        else:
            out_d = _norm(conv_d, stats_d, g_d, be_d, N_D, False,
                          W_proj[0], b_proj[0].reshape(1, D))[0]
            out_p = _norm(conv_p, stats_p, g_p, be_p, N_P, False,
                          W_proj[1], b_proj[1].reshape(1, D))[0]

    return (out_d, out_p)
